# baseline (speedup 1.0000x reference)
import jax
import jax.numpy as jnp
from jax.experimental import pallas as pl
from jax.experimental.pallas import tpu as pltpu

_ROWS = 8192
_COLS = 2048
_BLK = 512
_GRID = _ROWS // _BLK


def _body(x_ref, y_ref, m0_ref, m1_ref, o_ref, acc_ref):
    i = pl.program_id(0)

    @pl.when(i == 0)
    def _init():
        acc_ref[0] = 0.0
        acc_ref[1] = 0.0
        acc_ref[2] = 0.0
        acc_ref[3] = 0.0

    d = jnp.abs(x_ref[...] - y_ref[...])
    w0 = 1.0 - m0_ref[...].astype(jnp.float32)
    w1 = 1.0 - m1_ref[...].astype(jnp.float32)
    acc_ref[0] += jnp.sum(d * w0)
    acc_ref[1] += jnp.sum(w0)
    acc_ref[2] += jnp.sum(d * w1)
    acc_ref[3] += jnp.sum(w1)

    @pl.when(i == _GRID - 1)
    def _fin():
        o_ref[0] = acc_ref[0] / acc_ref[1] + acc_ref[2] / acc_ref[3]


def kernel(input, output, mask0, mask1):
    blocks_per_batch = 4096 // _BLK
    spec = pl.BlockSpec(
        (1, _BLK, _COLS),
        lambda i: (i // blocks_per_batch, i % blocks_per_batch, 0),
    )
    out = pl.pallas_call(
        _body,
        grid=(_GRID,),
        in_specs=[spec, spec, spec, spec],
        out_specs=pl.BlockSpec(memory_space=pltpu.SMEM),
        out_shape=jax.ShapeDtypeStruct((1,), jnp.float32),
        scratch_shapes=[pltpu.SMEM((4,), jnp.float32)],
        compiler_params=pltpu.CompilerParams(
            dimension_semantics=("arbitrary",),
        ),
    )(input, output, mask0, mask1)
    return out[0]


# speedup vs baseline: 1.7113x; 1.7113x over previous
import jax
import jax.numpy as jnp
from jax.experimental import pallas as pl
from jax.experimental.pallas import tpu as pltpu

_BATCH = 2
_ROWS = 4096
_COLS = 2048
_BLK = 512
_GRID = _BATCH * _ROWS // _BLK


def _body(x_ref, y_ref, w_ref, o_ref, acc_ref):
    i = pl.program_id(0)

    @pl.when(i == 0)
    def _init():
        acc_ref[0] = 0.0
        acc_ref[1] = 0.0
        acc_ref[2] = 0.0
        acc_ref[3] = 0.0

    d = jnp.abs(x_ref[...] - y_ref[...])
    t = w_ref[...].astype(jnp.int32)
    w0 = (t & 1).astype(jnp.float32)
    w1 = (t >> 1).astype(jnp.float32)
    acc_ref[0] += jnp.sum(d * w0)
    acc_ref[1] += jnp.sum(w0)
    acc_ref[2] += jnp.sum(d * w1)
    acc_ref[3] += jnp.sum(w1)

    @pl.when(i == _GRID - 1)
    def _fin():
        o_ref[0] = acc_ref[0] / acc_ref[1] + acc_ref[2] / acc_ref[3]


def kernel(input, output, mask0, mask1):
    w01 = (~mask0).astype(jnp.int8) + ((~mask1).astype(jnp.int8) << 1)

    blocks_per_batch = _ROWS // _BLK
    spec = pl.BlockSpec(
        (1, _BLK, _COLS),
        lambda i: (i // blocks_per_batch, i % blocks_per_batch, 0),
    )
    out = pl.pallas_call(
        _body,
        grid=(_GRID,),
        in_specs=[spec, spec, spec],
        out_specs=pl.BlockSpec(memory_space=pltpu.SMEM),
        out_shape=jax.ShapeDtypeStruct((1,), jnp.float32),
        scratch_shapes=[pltpu.SMEM((4,), jnp.float32)],
        compiler_params=pltpu.CompilerParams(
            dimension_semantics=("arbitrary",),
        ),
    )(input, output, w01)
    return out[0]
